# Initial kernel scaffold; baseline (speedup 1.0000x reference)
import jax
import jax.numpy as jnp
from jax.experimental import pallas as pl


def kernel(x, Ws, bs, W1, b1, W2, b2):
    raise NotImplementedError("write your pallas kernel here")


# speedup vs baseline: 6.1957x; 6.1957x over previous
import functools

import jax
import jax.numpy as jnp
from jax import lax
from jax.experimental import pallas as pl
from jax.experimental.pallas import tpu as pltpu

N = 8192
D = 768
E = 64
F = 1024
RB = 512
NRB = N // RB
TM = 128
NPAD = 16384
NT = NPAD // TM


def _router_body(x_ref, ws_ref, bs_ref, routes_ref, ranks_ref, pmax_ref,
                 counts_ref, colsum_ref, aux_ref):
    i = pl.program_id(0)

    @pl.when(i == 0)
    def _init():
        counts_ref[...] = jnp.zeros_like(counts_ref)
        colsum_ref[...] = jnp.zeros_like(colsum_ref)

    x = x_ref[...]
    logits = jnp.dot(x, ws_ref[...], preferred_element_type=jnp.float32)
    logits = logits + bs_ref[...]

    m = jnp.max(logits, axis=1, keepdims=True)
    p = jnp.exp(logits - m)
    s = jnp.sum(p, axis=1, keepdims=True)
    probs = p / s
    pmax = 1.0 / s[:, 0]

    iota_e = lax.broadcasted_iota(jnp.int32, (RB, E), 1)
    is_max = logits == m
    routes = jnp.min(jnp.where(is_max, iota_e, E), axis=1)

    onehot = (iota_e == routes[:, None]).astype(jnp.float32)

    ri = lax.broadcasted_iota(jnp.int32, (RB, RB), 0)
    rj = lax.broadcasted_iota(jnp.int32, (RB, RB), 1)
    tri = (rj <= ri).astype(jnp.float32)
    csum = jnp.dot(tri, onehot, preferred_element_type=jnp.float32)
    rank_in_blk = jnp.sum(csum * onehot, axis=1) - 1.0
    running = counts_ref[...]
    rank = rank_in_blk + jnp.sum(onehot * running, axis=1)

    cnt_blk = jnp.sum(onehot, axis=0, keepdims=True)
    counts_ref[...] = running + cnt_blk
    colsum_ref[...] = colsum_ref[...] + jnp.sum(probs, axis=0, keepdims=True)

    routes_ref[...] = routes.reshape(1, 1, RB)
    ranks_ref[...] = rank.astype(jnp.int32).reshape(1, 1, RB)
    pmax_ref[...] = pmax.reshape(1, 1, RB)

    @pl.when(i == NRB - 1)
    def _fin():
        aux_ref[...] = ((E / N) * jnp.sum(counts_ref[...] * colsum_ref[...])
                        ).reshape(1, 1)


def _router(xf, Ws, bs):
    return pl.pallas_call(
        _router_body,
        grid=(NRB,),
        in_specs=[
            pl.BlockSpec((RB, D), lambda i: (i, 0)),
            pl.BlockSpec((D, E), lambda i: (0, 0)),
            pl.BlockSpec((1, E), lambda i: (0, 0)),
        ],
        out_specs=[
            pl.BlockSpec((1, 1, RB), lambda i: (i, 0, 0)),
            pl.BlockSpec((1, 1, RB), lambda i: (i, 0, 0)),
            pl.BlockSpec((1, 1, RB), lambda i: (i, 0, 0)),
            pl.BlockSpec((1, E), lambda i: (0, 0)),
            pl.BlockSpec((1, E), lambda i: (0, 0)),
            pl.BlockSpec((1, 1), lambda i: (0, 0)),
        ],
        out_shape=[
            jax.ShapeDtypeStruct((NRB, 1, RB), jnp.int32),
            jax.ShapeDtypeStruct((NRB, 1, RB), jnp.int32),
            jax.ShapeDtypeStruct((NRB, 1, RB), jnp.float32),
            jax.ShapeDtypeStruct((1, E), jnp.float32),
            jax.ShapeDtypeStruct((1, E), jnp.float32),
            jax.ShapeDtypeStruct((1, 1), jnp.float32),
        ],
    )(xf, Ws, bs.reshape(1, E))


def _ffn_body(te_ref, x_ref, w1_ref, b1_ref, w2_ref, b2_ref, out_ref):
    x = x_ref[...]
    h = jnp.dot(x, w1_ref[0], preferred_element_type=jnp.float32)
    h = jnp.maximum(h + b1_ref[0], 0.0)
    o = jnp.dot(h, w2_ref[0], preferred_element_type=jnp.float32)
    out_ref[...] = o + b2_ref[0]


def _grouped_ffn(xs_padded, W1, b1, W2, b2, tile_expert):
    grid_spec = pltpu.PrefetchScalarGridSpec(
        num_scalar_prefetch=1,
        grid=(NT,),
        in_specs=[
            pl.BlockSpec((TM, D), lambda i, te: (i, 0)),
            pl.BlockSpec((1, D, F), lambda i, te: (te[i], 0, 0)),
            pl.BlockSpec((1, 1, F), lambda i, te: (te[i], 0, 0)),
            pl.BlockSpec((1, F, D), lambda i, te: (te[i], 0, 0)),
            pl.BlockSpec((1, 1, D), lambda i, te: (te[i], 0, 0)),
        ],
        out_specs=pl.BlockSpec((TM, D), lambda i, te: (i, 0)),
    )
    return pl.pallas_call(
        _ffn_body,
        grid_spec=grid_spec,
        out_shape=jax.ShapeDtypeStruct((NPAD, D), jnp.float32),
    )(tile_expert, xs_padded, W1, b1.reshape(E, 1, F), W2, b2.reshape(E, 1, D))


def kernel(x, Ws, bs, W1, b1, W2, b2):
    bsz, seq, d_model = x.shape
    xf = x.reshape(-1, d_model)

    routes3, ranks3, pmax3, counts_f, colsum, aux = _router(xf, Ws, bs)
    routes = routes3.reshape(N)
    ranks = ranks3.reshape(N)
    pmax = pmax3.reshape(N)

    counts = counts_f.reshape(E).astype(jnp.int32)
    padded = ((counts + TM - 1) // TM) * TM
    ends = jnp.cumsum(padded)
    starts = ends - padded
    dest = starts[routes] + ranks

    tile_expert = jnp.clip(
        jnp.searchsorted(ends, jnp.arange(NT, dtype=jnp.int32) * TM, side="right"),
        0, E - 1).astype(jnp.int32)

    xs_padded = jnp.zeros((NPAD, D), jnp.float32).at[dest].set(xf)
    out_padded = _grouped_ffn(xs_padded, W1, b1, W2, b2, tile_expert)

    final = out_padded[dest] * pmax[:, None]
    return final.reshape(bsz, seq, d_model), aux[0, 0]


# speedup vs baseline: 8.5049x; 1.3727x over previous
import functools

import jax
import jax.numpy as jnp
from jax import lax
from jax.experimental import pallas as pl
from jax.experimental.pallas import tpu as pltpu
from jax.experimental.pallas import tpu_sc as plsc

N = 8192
D = 768
E = 64
F = 1024
RB = 512
NRB = N // RB
TM = 128
NPAD = 16384
NT = NPAD // TM

SC_CORES = 2
SC_SUBCORES = 16
NW = SC_CORES * SC_SUBCORES
TPW = N // NW
CB = 64
NCH = TPW // CB


def _router_body(x_ref, ws_ref, bs_ref, routes_ref, ranks_ref, pmax_ref,
                 counts_ref, colsum_ref, aux_ref):
    i = pl.program_id(0)

    @pl.when(i == 0)
    def _init():
        counts_ref[...] = jnp.zeros_like(counts_ref)
        colsum_ref[...] = jnp.zeros_like(colsum_ref)

    x = x_ref[...]
    logits = jnp.dot(x, ws_ref[...], preferred_element_type=jnp.float32)
    logits = logits + bs_ref[...]

    m = jnp.max(logits, axis=1, keepdims=True)
    p = jnp.exp(logits - m)
    s = jnp.sum(p, axis=1, keepdims=True)
    probs = p / s
    pmax = 1.0 / s[:, 0]

    iota_e = lax.broadcasted_iota(jnp.int32, (RB, E), 1)
    is_max = logits == m
    routes = jnp.min(jnp.where(is_max, iota_e, E), axis=1)

    onehot = (iota_e == routes[:, None]).astype(jnp.float32)

    ri = lax.broadcasted_iota(jnp.int32, (RB, RB), 0)
    rj = lax.broadcasted_iota(jnp.int32, (RB, RB), 1)
    tri = (rj <= ri).astype(jnp.float32)
    csum = jnp.dot(tri, onehot, preferred_element_type=jnp.float32)
    rank_in_blk = jnp.sum(csum * onehot, axis=1) - 1.0
    running = counts_ref[...]
    rank = rank_in_blk + jnp.sum(onehot * running, axis=1)

    cnt_blk = jnp.sum(onehot, axis=0, keepdims=True)
    counts_ref[...] = running + cnt_blk
    colsum_ref[...] = colsum_ref[...] + jnp.sum(probs, axis=0, keepdims=True)

    routes_ref[...] = routes.reshape(1, 1, RB)
    ranks_ref[...] = rank.astype(jnp.int32).reshape(1, 1, RB)
    pmax_ref[...] = pmax.reshape(1, 1, RB)

    @pl.when(i == NRB - 1)
    def _fin():
        aux_ref[...] = ((E / N) * jnp.sum(counts_ref[...] * colsum_ref[...])
                        ).reshape(1, 1)


def _router(xf, Ws, bs):
    return pl.pallas_call(
        _router_body,
        grid=(NRB,),
        in_specs=[
            pl.BlockSpec((RB, D), lambda i: (i, 0)),
            pl.BlockSpec((D, E), lambda i: (0, 0)),
            pl.BlockSpec((1, E), lambda i: (0, 0)),
        ],
        out_specs=[
            pl.BlockSpec((1, 1, RB), lambda i: (i, 0, 0)),
            pl.BlockSpec((1, 1, RB), lambda i: (i, 0, 0)),
            pl.BlockSpec((1, 1, RB), lambda i: (i, 0, 0)),
            pl.BlockSpec((1, E), lambda i: (0, 0)),
            pl.BlockSpec((1, E), lambda i: (0, 0)),
            pl.BlockSpec((1, 1), lambda i: (0, 0)),
        ],
        out_shape=[
            jax.ShapeDtypeStruct((NRB, 1, RB), jnp.int32),
            jax.ShapeDtypeStruct((NRB, 1, RB), jnp.int32),
            jax.ShapeDtypeStruct((NRB, 1, RB), jnp.float32),
            jax.ShapeDtypeStruct((1, E), jnp.float32),
            jax.ShapeDtypeStruct((1, E), jnp.float32),
            jax.ShapeDtypeStruct((1, 1), jnp.float32),
        ],
    )(xf, Ws, bs.reshape(1, E))


_SC_MESH = plsc.VectorSubcoreMesh(
    core_axis_name="c", subcore_axis_name="s",
    num_cores=SC_CORES, num_subcores=SC_SUBCORES)


def _sc_scatter_body(xf, routes, ranks, starts, xs, dest,
                     rows_v, idx_v, rr_v, starts_v, sem):
    wid = lax.axis_index("s") * SC_CORES + lax.axis_index("c")
    wbase = wid * TPW
    pltpu.sync_copy(starts, starts_v)
    for c in range(NCH):
        base = wbase + c * CB
        pltpu.sync_copy(routes.at[pl.ds(base, CB)], rr_v.at[0])
        pltpu.sync_copy(ranks.at[pl.ds(base, CB)], rr_v.at[1])
        pltpu.sync_copy(xf.at[pl.ds(base, CB)], rows_v)
        for k in range(CB // 16):
            r = rr_v[0, pl.ds(k * 16, 16)]
            s_r = plsc.load_gather(starts_v, [r])
            idx_v[c, pl.ds(k * 16, 16)] = s_r + rr_v[1, pl.ds(k * 16, 16)]
        pltpu.async_copy(rows_v, xs.at[idx_v.at[c]], sem).wait()
        pltpu.sync_copy(idx_v.at[c], dest.at[pl.ds(base, CB)])


def _sc_scatter(xf, routes, ranks, starts):
    return pl.kernel(
        _sc_scatter_body,
        out_type=[
            jax.ShapeDtypeStruct((NPAD, D), jnp.float32),
            jax.ShapeDtypeStruct((N,), jnp.int32),
        ],
        mesh=_SC_MESH,
        compiler_params=pltpu.CompilerParams(needs_layout_passes=False),
        scratch_types=[
            pltpu.VMEM((CB, D), jnp.float32),
            pltpu.VMEM((NCH, CB), jnp.int32),
            pltpu.VMEM((2, CB), jnp.int32),
            pltpu.VMEM((E,), jnp.int32),
            pltpu.SemaphoreType.DMA,
        ],
    )(xf, routes, ranks, starts)


def _sc_gather_body(outp, dest, pmax, final, rows_v, idx_v, pmax_v, sem):
    wid = lax.axis_index("s") * SC_CORES + lax.axis_index("c")
    wbase = wid * TPW
    for c in range(NCH):
        base = wbase + c * CB
        pltpu.sync_copy(dest.at[pl.ds(base, CB)], idx_v)
        pltpu.sync_copy(pmax.at[pl.ds(base, CB)], pmax_v)
        pltpu.async_copy(outp.at[idx_v], rows_v, sem).wait()

        def _scale(k, carry):
            pv = plsc.load_gather(pmax_v, [jnp.zeros((16,), jnp.int32) + k])
            for j in range(D // 16):
                rows_v[k, pl.ds(j * 16, 16)] = rows_v[k, pl.ds(j * 16, 16)] * pv
            return carry

        lax.fori_loop(0, CB, _scale, 0)
        pltpu.sync_copy(rows_v, final.at[pl.ds(base, CB)])


def _sc_gather(outp, dest, pmax):
    return pl.kernel(
        _sc_gather_body,
        out_type=jax.ShapeDtypeStruct((N, D), jnp.float32),
        mesh=_SC_MESH,
        compiler_params=pltpu.CompilerParams(needs_layout_passes=False),
        scratch_types=[
            pltpu.VMEM((CB, D), jnp.float32),
            pltpu.VMEM((CB,), jnp.int32),
            pltpu.VMEM((CB,), jnp.float32),
            pltpu.SemaphoreType.DMA,
        ],
    )(outp, dest, pmax)


def _ffn_body(te_ref, ut_ref, x_ref, w1_ref, b1_ref, w2_ref, b2_ref, out_ref):
    i = pl.program_id(0)

    @pl.when(i < ut_ref[0])
    def _compute():
        x = x_ref[...]
        h = jnp.dot(x, w1_ref[0], preferred_element_type=jnp.float32)
        h = jnp.maximum(h + b1_ref[0], 0.0)
        o = jnp.dot(h, w2_ref[0], preferred_element_type=jnp.float32)
        out_ref[...] = o + b2_ref[0]


def _grouped_ffn(xs_padded, W1, b1, W2, b2, tile_expert, used_tiles):
    grid_spec = pltpu.PrefetchScalarGridSpec(
        num_scalar_prefetch=2,
        grid=(NT,),
        in_specs=[
            pl.BlockSpec((TM, D), lambda i, te, ut: (jnp.minimum(i, ut[0] - 1), 0)),
            pl.BlockSpec((1, D, F), lambda i, te, ut: (te[i], 0, 0)),
            pl.BlockSpec((1, 1, F), lambda i, te, ut: (te[i], 0, 0)),
            pl.BlockSpec((1, F, D), lambda i, te, ut: (te[i], 0, 0)),
            pl.BlockSpec((1, 1, D), lambda i, te, ut: (te[i], 0, 0)),
        ],
        out_specs=pl.BlockSpec(
            (TM, D), lambda i, te, ut: (jnp.minimum(i, ut[0] - 1), 0)),
    )
    return pl.pallas_call(
        _ffn_body,
        grid_spec=grid_spec,
        out_shape=jax.ShapeDtypeStruct((NPAD, D), jnp.float32),
    )(tile_expert, used_tiles, xs_padded, W1, b1.reshape(E, 1, F), W2,
      b2.reshape(E, 1, D))


def kernel(x, Ws, bs, W1, b1, W2, b2):
    bsz, seq, d_model = x.shape
    xf = x.reshape(-1, d_model)

    routes3, ranks3, pmax3, counts_f, colsum, aux = _router(xf, Ws, bs)
    routes = routes3.reshape(N)
    ranks = ranks3.reshape(N)
    pmax = pmax3.reshape(N)

    counts = counts_f.reshape(E).astype(jnp.int32)
    padded = ((counts + TM - 1) // TM) * TM
    ends = jnp.cumsum(padded)
    starts = (ends - padded).astype(jnp.int32)
    used_tiles = (ends[-1] // TM).astype(jnp.int32).reshape(1)

    tidx = jnp.arange(NT, dtype=jnp.int32)
    te = jnp.searchsorted(ends, tidx * TM, side="right").astype(jnp.int32)
    te_last = te[jnp.maximum(used_tiles[0] - 1, 0)]
    tile_expert = jnp.where(tidx < used_tiles[0], jnp.minimum(te, E - 1), te_last)

    xs_padded, dest = _sc_scatter(xf, routes, ranks, starts)
    out_padded = _grouped_ffn(xs_padded, W1, b1, W2, b2, tile_expert, used_tiles)
    final = _sc_gather(out_padded, dest, pmax)

    return final.reshape(bsz, seq, d_model), aux[0, 0]


# speedup vs baseline: 8.5401x; 1.0041x over previous
import functools

import jax
import jax.numpy as jnp
from jax import lax
from jax.experimental import pallas as pl
from jax.experimental.pallas import tpu as pltpu
from jax.experimental.pallas import tpu_sc as plsc

N = 8192
D = 768
E = 64
F = 1024
RB = 512
NRB = N // RB
TM = 128
NPAD = 16384
NT = NPAD // TM

SC_CORES = 2
SC_SUBCORES = 16
NW = SC_CORES * SC_SUBCORES
TPW = N // NW
CB = 64
NCH = TPW // CB


def _router_body(x_ref, ws_ref, bs_ref, routes_ref, ranks_ref, pmax_ref,
                 counts_ref, colsum_ref, aux_ref):
    i = pl.program_id(0)

    @pl.when(i == 0)
    def _init():
        counts_ref[...] = jnp.zeros_like(counts_ref)
        colsum_ref[...] = jnp.zeros_like(colsum_ref)

    x = x_ref[...]
    logits = jnp.dot(x, ws_ref[...], preferred_element_type=jnp.float32)
    logits = logits + bs_ref[...]

    m = jnp.max(logits, axis=1, keepdims=True)
    p = jnp.exp(logits - m)
    s = jnp.sum(p, axis=1, keepdims=True)
    probs = p / s
    pmax = 1.0 / s[:, 0]

    iota_e = lax.broadcasted_iota(jnp.int32, (RB, E), 1)
    is_max = logits == m
    routes = jnp.min(jnp.where(is_max, iota_e, E), axis=1)

    onehot = (iota_e == routes[:, None]).astype(jnp.float32)

    ri = lax.broadcasted_iota(jnp.int32, (RB, RB), 0)
    rj = lax.broadcasted_iota(jnp.int32, (RB, RB), 1)
    tri = (rj <= ri).astype(jnp.float32)
    csum = jnp.dot(tri, onehot, preferred_element_type=jnp.float32)
    rank_in_blk = jnp.sum(csum * onehot, axis=1) - 1.0
    running = counts_ref[...]
    rank = rank_in_blk + jnp.sum(onehot * running, axis=1)

    cnt_blk = jnp.sum(onehot, axis=0, keepdims=True)
    counts_ref[...] = running + cnt_blk
    colsum_ref[...] = colsum_ref[...] + jnp.sum(probs, axis=0, keepdims=True)

    routes_ref[...] = routes.reshape(1, 1, RB)
    ranks_ref[...] = rank.astype(jnp.int32).reshape(1, 1, RB)
    pmax_ref[...] = pmax.reshape(1, 1, RB)

    @pl.when(i == NRB - 1)
    def _fin():
        aux_ref[...] = ((E / N) * jnp.sum(counts_ref[...] * colsum_ref[...])
                        ).reshape(1, 1)


def _router(xf, Ws, bs):
    return pl.pallas_call(
        _router_body,
        grid=(NRB,),
        in_specs=[
            pl.BlockSpec((RB, D), lambda i: (i, 0)),
            pl.BlockSpec((D, E), lambda i: (0, 0)),
            pl.BlockSpec((1, E), lambda i: (0, 0)),
        ],
        out_specs=[
            pl.BlockSpec((1, 1, RB), lambda i: (i, 0, 0)),
            pl.BlockSpec((1, 1, RB), lambda i: (i, 0, 0)),
            pl.BlockSpec((1, 1, RB), lambda i: (i, 0, 0)),
            pl.BlockSpec((1, E), lambda i: (0, 0)),
            pl.BlockSpec((1, E), lambda i: (0, 0)),
            pl.BlockSpec((1, 1), lambda i: (0, 0)),
        ],
        out_shape=[
            jax.ShapeDtypeStruct((NRB, 1, RB), jnp.int32),
            jax.ShapeDtypeStruct((NRB, 1, RB), jnp.int32),
            jax.ShapeDtypeStruct((NRB, 1, RB), jnp.float32),
            jax.ShapeDtypeStruct((1, E), jnp.float32),
            jax.ShapeDtypeStruct((1, E), jnp.float32),
            jax.ShapeDtypeStruct((1, 1), jnp.float32),
        ],
    )(xf, Ws, bs.reshape(1, E))


_SC_MESH = plsc.VectorSubcoreMesh(
    core_axis_name="c", subcore_axis_name="s",
    num_cores=SC_CORES, num_subcores=SC_SUBCORES)


def _sc_scatter_body(xf, routes, ranks, starts, xs, dest,
                     rows_v, idx_v, rr_v, starts_v, sem):
    wid = lax.axis_index("s") * SC_CORES + lax.axis_index("c")
    wbase = wid * TPW
    pltpu.sync_copy(starts, starts_v)
    for c in range(NCH):
        base = wbase + c * CB
        pltpu.sync_copy(routes.at[pl.ds(base, CB)], rr_v.at[0])
        pltpu.sync_copy(ranks.at[pl.ds(base, CB)], rr_v.at[1])
        pltpu.sync_copy(xf.at[pl.ds(base, CB)], rows_v)
        for k in range(CB // 16):
            r = rr_v[0, pl.ds(k * 16, 16)]
            s_r = plsc.load_gather(starts_v, [r])
            idx_v[c, pl.ds(k * 16, 16)] = s_r + rr_v[1, pl.ds(k * 16, 16)]
        pltpu.async_copy(rows_v, xs.at[idx_v.at[c]], sem).wait()
        pltpu.sync_copy(idx_v.at[c], dest.at[pl.ds(base, CB)])


def _sc_scatter(xf, routes, ranks, starts):
    return pl.kernel(
        _sc_scatter_body,
        out_type=[
            jax.ShapeDtypeStruct((NPAD, D), jnp.float32),
            jax.ShapeDtypeStruct((N,), jnp.int32),
        ],
        mesh=_SC_MESH,
        compiler_params=pltpu.CompilerParams(needs_layout_passes=False),
        scratch_types=[
            pltpu.VMEM((CB, D), jnp.float32),
            pltpu.VMEM((NCH, CB), jnp.int32),
            pltpu.VMEM((2, CB), jnp.int32),
            pltpu.VMEM((E,), jnp.int32),
            pltpu.SemaphoreType.DMA,
        ],
    )(xf, routes, ranks, starts)


def _sc_gather_body(outp, dest, pmax, final, rows_v, idx_v, pmax_v, sem):
    wid = lax.axis_index("s") * SC_CORES + lax.axis_index("c")
    wbase = wid * TPW
    for c in range(NCH):
        base = wbase + c * CB
        pltpu.sync_copy(dest.at[pl.ds(base, CB)], idx_v)
        pltpu.sync_copy(pmax.at[pl.ds(base, CB)], pmax_v)
        pltpu.async_copy(outp.at[idx_v], rows_v, sem).wait()

        def _scale(k, carry):
            pv = plsc.load_gather(pmax_v, [jnp.zeros((16,), jnp.int32) + k])
            for j in range(D // 16):
                rows_v[k, pl.ds(j * 16, 16)] = rows_v[k, pl.ds(j * 16, 16)] * pv
            return carry

        lax.fori_loop(0, CB, _scale, 0)
        pltpu.sync_copy(rows_v, final.at[pl.ds(base, CB)])


def _sc_gather(outp, dest, pmax):
    return pl.kernel(
        _sc_gather_body,
        out_type=jax.ShapeDtypeStruct((N, D), jnp.float32),
        mesh=_SC_MESH,
        compiler_params=pltpu.CompilerParams(needs_layout_passes=False),
        scratch_types=[
            pltpu.VMEM((CB, D), jnp.float32),
            pltpu.VMEM((CB,), jnp.int32),
            pltpu.VMEM((CB,), jnp.float32),
            pltpu.SemaphoreType.DMA,
        ],
    )(outp, dest, pmax)


def _ffn_body(te_ref, ut_ref, x_ref, w1_ref, b1_ref, w2_ref, b2_ref, out_ref):
    i = pl.program_id(0)

    @pl.when(i < ut_ref[0])
    def _compute():
        x = x_ref[...].astype(jnp.bfloat16)
        w1 = w1_ref[0].astype(jnp.bfloat16)
        h = jnp.dot(x, w1, preferred_element_type=jnp.float32)
        h = jnp.maximum(h + b1_ref[0], 0.0)
        w2 = w2_ref[0].astype(jnp.bfloat16)
        o = jnp.dot(h.astype(jnp.bfloat16), w2, preferred_element_type=jnp.float32)
        out_ref[...] = o + b2_ref[0]


def _grouped_ffn(xs_padded, W1, b1, W2, b2, tile_expert, used_tiles):
    grid_spec = pltpu.PrefetchScalarGridSpec(
        num_scalar_prefetch=2,
        grid=(NT,),
        in_specs=[
            pl.BlockSpec((TM, D), lambda i, te, ut: (jnp.minimum(i, ut[0] - 1), 0)),
            pl.BlockSpec((1, D, F), lambda i, te, ut: (te[i], 0, 0)),
            pl.BlockSpec((1, 1, F), lambda i, te, ut: (te[i], 0, 0)),
            pl.BlockSpec((1, F, D), lambda i, te, ut: (te[i], 0, 0)),
            pl.BlockSpec((1, 1, D), lambda i, te, ut: (te[i], 0, 0)),
        ],
        out_specs=pl.BlockSpec(
            (TM, D), lambda i, te, ut: (jnp.minimum(i, ut[0] - 1), 0)),
    )
    return pl.pallas_call(
        _ffn_body,
        grid_spec=grid_spec,
        out_shape=jax.ShapeDtypeStruct((NPAD, D), jnp.float32),
    )(tile_expert, used_tiles, xs_padded, W1, b1.reshape(E, 1, F), W2,
      b2.reshape(E, 1, D))


def kernel(x, Ws, bs, W1, b1, W2, b2):
    bsz, seq, d_model = x.shape
    xf = x.reshape(-1, d_model)

    routes3, ranks3, pmax3, counts_f, colsum, aux = _router(xf, Ws, bs)
    routes = routes3.reshape(N)
    ranks = ranks3.reshape(N)
    pmax = pmax3.reshape(N)

    counts = counts_f.reshape(E).astype(jnp.int32)
    padded = ((counts + TM - 1) // TM) * TM
    ends = jnp.cumsum(padded)
    starts = (ends - padded).astype(jnp.int32)
    used_tiles = (ends[-1] // TM).astype(jnp.int32).reshape(1)

    tidx = jnp.arange(NT, dtype=jnp.int32)
    te = jnp.searchsorted(ends, tidx * TM, side="right").astype(jnp.int32)
    te_last = te[jnp.maximum(used_tiles[0] - 1, 0)]
    tile_expert = jnp.where(tidx < used_tiles[0], jnp.minimum(te, E - 1), te_last)

    xs_padded, dest = _sc_scatter(xf, routes, ranks, starts)
    out_padded = _grouped_ffn(xs_padded, W1, b1, W2, b2, tile_expert, used_tiles)
    final = _sc_gather(out_padded, dest, pmax)

    return final.reshape(bsz, seq, d_model), aux[0, 0]
